# baseline (speedup 1.0000x reference)
import jax
import jax.numpy as jnp
from jax.experimental import pallas as pl


_LANES = 128
_ROWS_PER_CH = (64 * 64) // _LANES
_BLOCK_ROWS = 4096


def _swap_body(x1_ref, x2_ref, o1_ref, o2_ref):
    row = jax.lax.broadcasted_iota(jnp.int32, x1_ref.shape, 0)
    mask = ((row // _ROWS_PER_CH) % 2) == 0
    a = x1_ref[...]
    b = x2_ref[...]
    o1_ref[...] = jnp.where(mask, b, a)
    o2_ref[...] = jnp.where(mask, a, b)


def kernel(x1, x2):
    N, c, h, w = x1.shape
    rows = (N * c * h * w) // _LANES
    a = x1.reshape(rows, _LANES)
    b = x2.reshape(rows, _LANES)
    spec = pl.BlockSpec((_BLOCK_ROWS, _LANES), lambda i: (i, 0))
    o1, o2 = pl.pallas_call(
        _swap_body,
        grid=(rows // _BLOCK_ROWS,),
        in_specs=[spec, spec],
        out_specs=[spec, spec],
        out_shape=[
            jax.ShapeDtypeStruct((rows, _LANES), x1.dtype),
            jax.ShapeDtypeStruct((rows, _LANES), x2.dtype),
        ],
    )(a, b)
    return (o1.reshape(N, c, h, w), o2.reshape(N, c, h, w))


# speedup vs baseline: 1.0106x; 1.0106x over previous
import jax
import jax.numpy as jnp
from jax.experimental import pallas as pl


_LANES = 128
_ROWS_PER_CH = (64 * 64) // _LANES
_BLOCK_ROWS = 8192


def _swap_body(x1_ref, x2_ref, o1_ref, o2_ref):
    row = jax.lax.broadcasted_iota(jnp.int32, x1_ref.shape, 0)
    mask = ((row // _ROWS_PER_CH) % 2) == 0
    a = x1_ref[...]
    b = x2_ref[...]
    o1_ref[...] = jnp.where(mask, b, a)
    o2_ref[...] = jnp.where(mask, a, b)


def kernel(x1, x2):
    N, c, h, w = x1.shape
    rows = (N * c * h * w) // _LANES
    a = x1.reshape(rows, _LANES)
    b = x2.reshape(rows, _LANES)
    spec = pl.BlockSpec((_BLOCK_ROWS, _LANES), lambda i: (i, 0))
    o1, o2 = pl.pallas_call(
        _swap_body,
        grid=(rows // _BLOCK_ROWS,),
        in_specs=[spec, spec],
        out_specs=[spec, spec],
        out_shape=[
            jax.ShapeDtypeStruct((rows, _LANES), x1.dtype),
            jax.ShapeDtypeStruct((rows, _LANES), x2.dtype),
        ],
    )(a, b)
    return (o1.reshape(N, c, h, w), o2.reshape(N, c, h, w))


# speedup vs baseline: 1.3442x; 1.3301x over previous
import functools

import jax
import jax.numpy as jnp
from jax import lax
from jax.experimental import pallas as pl
from jax.experimental.pallas import tpu as pltpu
from jax.experimental.pallas import tpu_sc as plsc


_N = 8
_CPAIRS = 96
_H = 64
_W = 64
_NWORKERS = 32
_PAIRS_PER_WORKER = (_N * _CPAIRS) // _NWORKERS
_CHUNK = 1
_NCHUNKS = _PAIRS_PER_WORKER // _CHUNK
_NSLOTS = 3


def _make_sc_kernel(dtype):
    mesh = plsc.VectorSubcoreMesh(core_axis_name="c", subcore_axis_name="s")
    out_sds = jax.ShapeDtypeStruct((_N, _CPAIRS, 2, _H, _W), dtype)
    buf_t = pltpu.VMEM((_NSLOTS, _CHUNK, 2, _H, _W), dtype)

    @functools.partial(
        pl.kernel,
        mesh=mesh,
        out_type=[out_sds, out_sds],
        scratch_types=[
            buf_t,
            buf_t,
            pltpu.SemaphoreType.DMA((_NSLOTS,)),
            pltpu.SemaphoreType.DMA((_NSLOTS,)),
        ],
    )
    def sc_exchange(x1_hbm, x2_hbm, o1_hbm, o2_hbm, buf_a, buf_b, sem_in, sem_out):
        wid = lax.axis_index("s") * 2 + lax.axis_index("c")
        workers_per_sample = _CPAIRS // _PAIRS_PER_WORKER
        n = wid // workers_per_sample
        p0 = (wid % workers_per_sample) * _PAIRS_PER_WORKER

        def in_copies(k, slot):
            sl = (n, pl.ds(p0 + k * _CHUNK, _CHUNK))
            return (
                pltpu.make_async_copy(x1_hbm.at[sl], buf_a.at[slot], sem_in.at[slot]),
                pltpu.make_async_copy(x2_hbm.at[sl], buf_b.at[slot], sem_in.at[slot]),
            )

        def out_copies(k, slot):
            sl = (n, pl.ds(p0 + k * _CHUNK, _CHUNK))
            return (
                pltpu.make_async_copy(buf_b.at[slot, :, 0], o1_hbm.at[sl + (0,)], sem_out.at[slot]),
                pltpu.make_async_copy(buf_a.at[slot, :, 1], o1_hbm.at[sl + (1,)], sem_out.at[slot]),
                pltpu.make_async_copy(buf_a.at[slot, :, 0], o2_hbm.at[sl + (0,)], sem_out.at[slot]),
                pltpu.make_async_copy(buf_b.at[slot, :, 1], o2_hbm.at[sl + (1,)], sem_out.at[slot]),
            )

        for k in range(_NCHUNKS):
            slot = k % _NSLOTS
            if k >= _NSLOTS:
                for cp in out_copies(k - _NSLOTS, slot):
                    cp.wait()
            for cp in in_copies(k, slot):
                cp.start()
            for cp in in_copies(k, slot):
                cp.wait()
            for cp in out_copies(k, slot):
                cp.start()

        for k in range(_NCHUNKS - _NSLOTS, _NCHUNKS):
            for cp in out_copies(k, k % _NSLOTS):
                cp.wait()

    return sc_exchange


def kernel(x1, x2):
    N, c, h, w = x1.shape
    a = x1.reshape(N, c // 2, 2, h, w)
    b = x2.reshape(N, c // 2, 2, h, w)
    o1, o2 = _make_sc_kernel(x1.dtype)(a, b)
    return (o1.reshape(N, c, h, w), o2.reshape(N, c, h, w))
